# baseline (speedup 1.0000x reference)
import functools

import jax
import jax.numpy as jnp
from jax import lax
from jax.experimental import pallas as pl
from jax.experimental.pallas import tpu as pltpu
from jax.experimental.pallas import tpu_sc as plsc

N_NODES = 10000
N_EDGES = 320000
D_FEAT = 128
N_REL = 4

NC = 2
HW = D_FEAT // NC
NS = 16
KB = 128
NB = 160
NR = 4
NSTAGE = 4
SB = NB // NSTAGE
EPW = NB * KB
E_PAD = NS * EPW
ACC_ROWS = 40448
TRASH_ROW = N_REL * N_NODES
ZROWS = ACC_ROWS // NS

BN = 1000


XROWS = N_NODES // NS


def _sc_body(xh_hbm, src_hbm, scat_hbm, zeros_hbm, acc_hbm,
             sidx, didx, rows, accs, x_sh, semg):
    c = lax.axis_index("c")
    s = lax.axis_index("s")

    def gat(b, t):
        pltpu.async_copy(x_sh.at[sidx.at[b]], rows.at[t], semg[t])

    def gat_wait(b, t):
        pltpu.make_async_copy(x_sh.at[sidx.at[b]], rows.at[t],
                              semg[t]).wait()

    pltpu.sync_copy(xh_hbm.at[pl.ds(s * XROWS, XROWS), pl.ds(c * HW, HW)],
                    x_sh.at[pl.ds(s * XROWS, XROWS)])
    pltpu.sync_copy(zeros_hbm, accs.at[pl.ds(s * ZROWS, ZROWS)])
    plsc.subcore_barrier()

    for h in range(NSTAGE):
        pltpu.sync_copy(src_hbm.at[s, h], sidx)
        pltpu.sync_copy(scat_hbm.at[s, h], didx)

        for t in range(NR):
            gat(t, t)

        def body(g, carry):
            b = NR * g
            for t in range(NR):
                gat_wait(b + t, t)
                pltpu.sync_copy(rows.at[t], accs.at[didx.at[b + t]],
                                add=True)

                @pl.when(b + t + NR < SB)
                def _():
                    gat(b + t + NR, t)
            return carry

        lax.fori_loop(0, SB // NR, body, 0)

    plsc.subcore_barrier()
    pltpu.sync_copy(accs.at[pl.ds(s * ZROWS, ZROWS)],
                    acc_hbm.at[pl.ds(s * ZROWS, ZROWS), pl.ds(c * HW, HW)])


@functools.cache
def _sc_scatter():
    return pl.kernel(
        _sc_body,
        out_type=jax.ShapeDtypeStruct((ACC_ROWS, D_FEAT), jnp.bfloat16),
        mesh=plsc.VectorSubcoreMesh(core_axis_name="c", subcore_axis_name="s"),
        scratch_types=[
            pltpu.VMEM((SB, KB), jnp.int32),
            pltpu.VMEM((SB, KB), jnp.int32),
            pltpu.VMEM((NR, KB, HW), jnp.bfloat16),
            pltpu.VMEM_SHARED((ACC_ROWS, HW), jnp.bfloat16),
            pltpu.VMEM_SHARED((N_NODES, HW), jnp.bfloat16),
            [pltpu.SemaphoreType.DMA] * NR,
        ],
        compiler_params=pltpu.CompilerParams(use_tc_tiling_on_sc=False),
    )


def _tc_body(x_ref, acc_ref, ws_ref, wn_ref, b_ref, out_ref):
    r = pl.program_id(1)

    @pl.when(r == 0)
    def _():
        out_ref[...] = jnp.zeros_like(out_ref)

    conv = (jnp.dot(x_ref[...], ws_ref[0], preferred_element_type=jnp.float32)
            + jnp.dot(acc_ref[...], wn_ref[0],
                      preferred_element_type=jnp.float32)
            + b_ref[0])
    out_ref[...] += jnp.tanh(conv)


_tc_dense = pl.pallas_call(
    _tc_body,
    grid=(N_NODES // BN, N_REL),
    in_specs=[
        pl.BlockSpec((BN, D_FEAT), lambda nb, r: (nb, 0)),
        pl.BlockSpec((BN, D_FEAT),
                     lambda nb, r: (r * (N_NODES // BN) + nb, 0)),
        pl.BlockSpec((1, D_FEAT, D_FEAT), lambda nb, r: (r, 0, 0)),
        pl.BlockSpec((1, D_FEAT, D_FEAT), lambda nb, r: (r, 0, 0)),
        pl.BlockSpec((1, 1, D_FEAT), lambda nb, r: (r, 0, 0)),
    ],
    out_specs=pl.BlockSpec((BN, D_FEAT), lambda nb, r: (nb, 0)),
    out_shape=jax.ShapeDtypeStruct((N_NODES, D_FEAT), jnp.float32),
    compiler_params=pltpu.CompilerParams(
        dimension_semantics=("arbitrary", "arbitrary")),
)


def kernel(x, edge_index, edge_attr, W_self, W_neigh, b):
    src = edge_index[0]
    dst = edge_index[1]
    xh = x.astype(jnp.bfloat16)
    pad = E_PAD - N_EDGES
    srcp = jnp.concatenate([src, jnp.zeros((pad,), jnp.int32)])
    srcp = srcp.reshape(NS, NSTAGE, SB, KB)
    scat = jnp.concatenate(
        [edge_attr * N_NODES + dst,
         jnp.full((pad,), TRASH_ROW, jnp.int32)]).reshape(NS, NSTAGE, SB, KB)
    zeros_z = jnp.zeros((ZROWS, HW), jnp.bfloat16)

    acc = _sc_scatter()(xh, srcp, scat, zeros_z)

    return _tc_dense(xh, acc, W_self.astype(jnp.bfloat16),
                     W_neigh.astype(jnp.bfloat16),
                     b.reshape(N_REL, 1, D_FEAT))


# speedup vs baseline: 1.0353x; 1.0353x over previous
import functools

import jax
import jax.numpy as jnp
from jax import lax
from jax.experimental import pallas as pl
from jax.experimental.pallas import tpu as pltpu
from jax.experimental.pallas import tpu_sc as plsc

N_NODES = 10000
N_EDGES = 320000
D_FEAT = 128
N_REL = 4

NC = 2
HW = D_FEAT // NC
NS = 16
KB = 128
NB = 160
NR = 4
NSTAGE = 4
SB = NB // NSTAGE
EPW = NB * KB
E_PAD = NS * EPW
ACC_ROWS = 40448
TRASH_ROW = N_REL * N_NODES
ZROWS = ACC_ROWS // NS

BN = 1000


XROWS = N_NODES // NS


def _sc_body(xh_hbm, src_hbm, scat_hbm, zeros_hbm, acc_hbm,
             sidx, didx, rows, accs, x_sh, semg):
    c = lax.axis_index("c")
    s = lax.axis_index("s")

    def gat(b, t):
        pltpu.async_copy(x_sh.at[sidx.at[b]], rows.at[t], semg[t])

    def gat_wait(b, t):
        pltpu.make_async_copy(x_sh.at[sidx.at[b]], rows.at[t],
                              semg[t]).wait()

    pltpu.sync_copy(xh_hbm.at[pl.ds(s * XROWS, XROWS), pl.ds(c * HW, HW)],
                    x_sh.at[pl.ds(s * XROWS, XROWS)])
    pltpu.sync_copy(zeros_hbm, accs.at[pl.ds(s * ZROWS, ZROWS)])
    plsc.subcore_barrier()

    for h in range(NSTAGE):
        pltpu.sync_copy(src_hbm.at[s, h], sidx)
        pltpu.sync_copy(scat_hbm.at[s, h], didx)

        for t in range(NR):
            gat(t, t)

        def body(g, carry):
            b = NR * g
            for t in range(NR):
                gat_wait(b + t, t)
                pltpu.sync_copy(rows.at[t], accs.at[didx.at[b + t]],
                                add=True)

                @pl.when(b + t + NR < SB)
                def _():
                    gat(b + t + NR, t)
            return carry

        lax.fori_loop(0, SB // NR, body, 0)

    plsc.subcore_barrier()
    pltpu.sync_copy(accs.at[pl.ds(s * ZROWS, ZROWS)],
                    acc_hbm.at[pl.ds(s * ZROWS, ZROWS), pl.ds(c * HW, HW)])


@functools.cache
def _sc_scatter():
    return pl.kernel(
        _sc_body,
        out_type=jax.ShapeDtypeStruct((ACC_ROWS, D_FEAT), jnp.bfloat16),
        mesh=plsc.VectorSubcoreMesh(core_axis_name="c", subcore_axis_name="s"),
        scratch_types=[
            pltpu.VMEM((SB, KB), jnp.int32),
            pltpu.VMEM((SB, KB), jnp.int32),
            pltpu.VMEM((NR, KB, HW), jnp.bfloat16),
            pltpu.VMEM_SHARED((ACC_ROWS, HW), jnp.bfloat16),
            pltpu.VMEM_SHARED((N_NODES, HW), jnp.bfloat16),
            [pltpu.SemaphoreType.DMA] * NR,
        ],
        compiler_params=pltpu.CompilerParams(use_tc_tiling_on_sc=False),
    )


def _tc_body(x_ref, acc_ref, ws_ref, wn_ref, b_ref, out_ref):
    r = pl.program_id(1)

    @pl.when(r == 0)
    def _():
        out_ref[...] = jnp.zeros_like(out_ref)

    conv = (jnp.dot(x_ref[...], ws_ref[0], preferred_element_type=jnp.float32)
            + jnp.dot(acc_ref[...], wn_ref[0],
                      preferred_element_type=jnp.float32)
            + b_ref[0])
    out_ref[...] += jnp.tanh(conv)


_tc_dense = pl.pallas_call(
    _tc_body,
    grid=(N_NODES // BN, N_REL),
    in_specs=[
        pl.BlockSpec((BN, D_FEAT), lambda nb, r: (nb, 0)),
        pl.BlockSpec((BN, D_FEAT),
                     lambda nb, r: (r * (N_NODES // BN) + nb, 0)),
        pl.BlockSpec((1, D_FEAT, D_FEAT), lambda nb, r: (r, 0, 0)),
        pl.BlockSpec((1, D_FEAT, D_FEAT), lambda nb, r: (r, 0, 0)),
        pl.BlockSpec((1, 1, D_FEAT), lambda nb, r: (r, 0, 0)),
    ],
    out_specs=pl.BlockSpec((BN, D_FEAT), lambda nb, r: (nb, 0)),
    out_shape=jax.ShapeDtypeStruct((N_NODES, D_FEAT), jnp.float32),
    compiler_params=pltpu.CompilerParams(
        dimension_semantics=("arbitrary", "arbitrary")),
)


def kernel(x, edge_index, edge_attr, W_self, W_neigh, b):
    src = edge_index[0]
    dst = edge_index[1]
    xh = x.astype(jnp.bfloat16)
    pad = E_PAD - N_EDGES
    srcp = jnp.concatenate([src, jnp.zeros((pad,), jnp.int32)])
    srcp = srcp.reshape(NS, NSTAGE, SB, KB)
    scat = jnp.concatenate(
        [edge_attr * N_NODES + dst,
         jnp.full((pad,), TRASH_ROW, jnp.int32)]).reshape(NS, NSTAGE, SB, KB)
    zeros_z = jnp.zeros((ZROWS, HW), jnp.bfloat16)

    acc = _sc_scatter()(xh, srcp, scat, zeros_z)

    return _tc_dense(x, acc, W_self, W_neigh.astype(jnp.bfloat16),
                     b.reshape(N_REL, 1, D_FEAT))
